# baseline (speedup 1.0000x reference)
import functools

import jax
import jax.numpy as jnp
from jax import lax
from jax.experimental import pallas as pl
from jax.experimental.pallas import tpu as pltpu
from jax.experimental.pallas import tpu_sc as plsc

N = 10000
D = 128
NEIG = 32
NPAD = 10240
TRASH = N
NTILES = 16
NW = 32
HALF = NPAD // 2
HROWS = HALF + 8

E = 320000
B = 128
HCH = 160
HPW = HCH * B
E_HOP = NTILES * HPW
ROWBLK = 400
GRID = N // ROWBLK

_mesh = plsc.VectorSubcoreMesh(core_axis_name="c", subcore_axis_name="s")


@functools.partial(
    pl.kernel,
    mesh=_mesh,
    out_type=jax.ShapeDtypeStruct((NPAD, D), jnp.float32),
    scratch_types=[
        pltpu.VMEM((HCH, B), jnp.int32),
        pltpu.VMEM((B, D), jnp.float32),
        pltpu.VMEM_SHARED((HROWS, D), jnp.float32),
    ],
)
def _sc_degree(dst_hbm, deg_out, dst_v, const_v, acc):
    c = lax.axis_index("c")
    s = lax.axis_index("s")

    def fill(val):
        def row(i, _):
            for k in range(D // 16):
                const_v[i, pl.ds(k * 16, 16)] = jnp.full((16,), val,
                                                         jnp.float32)
            return 0
        lax.fori_loop(0, B, row, 0)

    fill(0.0)
    base = pl.multiple_of(s * (HALF // NTILES), 8)
    for k in range(HALF // NTILES // B):
        pltpu.sync_copy(const_v, acc.at[pl.ds(base + k * B, B)])
    for k in range(HALF // NTILES % B // 64):
        pltpu.sync_copy(const_v.at[pl.ds(0, 64)],
                        acc.at[pl.ds(base + (HALF // NTILES // B) * B
                                     + k * 64, 64)])

    @pl.when(s == 0)
    def _():
        pltpu.sync_copy(const_v.at[pl.ds(0, 8)], acc.at[pl.ds(HALF, 8)])

    fill(1.0)

    pltpu.sync_copy(dst_hbm.at[s], dst_v)

    lo = c * HALF
    def remap(j, _):
        r = j // (B // 16)
        k = j % (B // 16)
        v = dst_v[r, pl.ds(k * 16, 16)]
        local = v - lo
        ok = jnp.logical_and(local >= 0, local < HALF)
        dst_v[r, pl.ds(k * 16, 16)] = jnp.where(ok, local, HALF)
        return 0
    lax.fori_loop(0, HCH * (B // 16), remap, 0)
    plsc.subcore_barrier()

    def body(j, _):
        pltpu.sync_copy(const_v, acc.at[dst_v.at[j]], add=True)
        return 0
    lax.fori_loop(0, HCH, body, 0)
    plsc.subcore_barrier()

    obase = pl.multiple_of(lo + base, 8)
    pltpu.sync_copy(acc.at[pl.ds(base, HALF // NTILES)],
                    deg_out.at[pl.ds(obase, HALF // NTILES)])


@functools.partial(
    pl.kernel,
    mesh=_mesh,
    out_type=jax.ShapeDtypeStruct((NPAD, D), jnp.float32),
    scratch_types=[
        pltpu.VMEM((HCH, B), jnp.int32),
        pltpu.VMEM((HCH, B), jnp.int32),
        pltpu.VMEM((B, D), jnp.float32),
        pltpu.VMEM((B, D), jnp.float32),
        pltpu.VMEM_SHARED((HROWS, D), jnp.float32),
        pltpu.SemaphoreType.DMA,
        pltpu.SemaphoreType.DMA,
    ],
)
def _sc_hop(y_hbm, src_hbm, dst_hbm, a_out, src_v, dst_v, r0, r1,
            acc, s0, s1):
    c = lax.axis_index("c")
    s = lax.axis_index("s")
    bufs = (r0, r1)
    sems = (s0, s1)

    def zrow(i, _):
        for k in range(D // 16):
            r0[i, pl.ds(k * 16, 16)] = jnp.zeros((16,), jnp.float32)
        return 0
    lax.fori_loop(0, B, zrow, 0)
    base = pl.multiple_of(s * (HALF // NTILES), 8)
    for k in range(HALF // NTILES // B):
        pltpu.sync_copy(r0, acc.at[pl.ds(base + k * B, B)])
    for k in range(HALF // NTILES % B // 64):
        pltpu.sync_copy(r0.at[pl.ds(0, 64)],
                        acc.at[pl.ds(base + (HALF // NTILES // B) * B
                                     + k * 64, 64)])

    @pl.when(s == 0)
    def _():
        pltpu.sync_copy(r0.at[pl.ds(0, 8)], acc.at[pl.ds(HALF, 8)])

    pltpu.sync_copy(src_hbm.at[s], src_v)
    pltpu.sync_copy(dst_hbm.at[s], dst_v)

    lo = c * HALF
    def remap(j, _):
        r = j // (B // 16)
        k = j % (B // 16)
        v = dst_v[r, pl.ds(k * 16, 16)]
        local = v - lo
        ok = jnp.logical_and(local >= 0, local < HALF)
        dst_v[r, pl.ds(k * 16, 16)] = jnp.where(ok, local, HALF)
        return 0
    lax.fori_loop(0, HCH * (B // 16), remap, 0)
    plsc.subcore_barrier()

    def body(j, _):
        pltpu.async_copy(y_hbm.at[src_v.at[j]], bufs[0], sems[0]).wait()
        pltpu.sync_copy(bufs[0], acc.at[dst_v.at[j]], add=True)
        return 0
    lax.fori_loop(0, HCH, body, 0)
    plsc.subcore_barrier()

    obase = pl.multiple_of(lo + base, 8)
    pltpu.sync_copy(acc.at[pl.ds(base, HALF // NTILES)],
                    a_out.at[pl.ds(obase, HALF // NTILES)])


def _tc_prep_body(f_ref, dp_ref, ev_ref, y0_ref, s_ref, dinv_ref, g_ref):
    i = pl.program_id(0)
    dp = dp_ref[...]
    deg = dp[:, 0]
    degc = jnp.maximum(deg, 1.0)
    s = lax.rsqrt(degc)
    s_ref[...] = s[:, None]
    dinv_ref[...] = (1.0 / degc)[:, None]
    x = f_ref[...]
    y0_ref[...] = x * s[:, None]

    @pl.when(i == 0)
    def _():
        g_ref[...] = jnp.zeros_like(g_ref)

    g_ref[...] += lax.dot_general(ev_ref[...], x, (((0,), (0,)), ((), ())),
                                  preferred_element_type=jnp.float32)


def _tc_mid_body(a_ref, dinv_ref, g_ref, evals_ref, w1_ref, b1_ref, w2_ref,
                 b2_ref, w3_ref, b3_ref, wlin_ref, y1_ref, g2_ref):
    a = a_ref[...]
    di = dinv_ref[...]
    y1_ref[...] = -(a * di)

    ev = evals_ref[...].reshape(NEIG, 1)
    t = jnp.maximum(jnp.dot(ev, w1_ref[...],
                            preferred_element_type=jnp.float32)
                    + b1_ref[...][None, :], 0.0)
    t = jnp.maximum(jnp.dot(t, w2_ref[...],
                            preferred_element_type=jnp.float32)
                    + b2_ref[...][None, :], 0.0)
    filt = jnp.dot(t, w3_ref[...],
                   preferred_element_type=jnp.float32) + b3_ref[...][None, :]
    g2_ref[...] = jnp.dot(filt * g_ref[...], wlin_ref[3 * D:, :],
                          preferred_element_type=jnp.float32)


def _tc_final_body(f_ref, a1_ref, a2_ref, s_ref, ev_ref, g2_ref, wlin_ref,
                   blin_ref, out_ref):
    x = f_ref[...]
    sc = s_ref[...]
    x1 = -sc * a1_ref[...]
    x2 = -2.0 * sc * a2_ref[...] - x
    wl = wlin_ref[...]
    acc = jnp.dot(x, wl[:D], preferred_element_type=jnp.float32)
    acc += jnp.dot(x1, wl[D:2 * D], preferred_element_type=jnp.float32)
    acc += jnp.dot(x2, wl[2 * D:3 * D], preferred_element_type=jnp.float32)
    acc += jnp.dot(ev_ref[...], g2_ref[...],
                   preferred_element_type=jnp.float32)
    out_ref[...] = jnp.maximum(acc + blin_ref[...][None, :], 0.0)


def _const_spec(shape):
    n = len(shape)
    return pl.BlockSpec(shape, lambda i: (0,) * n)


_tc_prep = pl.pallas_call(
    _tc_prep_body,
    grid=(GRID,),
    in_specs=[
        pl.BlockSpec((ROWBLK, D), lambda i: (i, 0)),
        pl.BlockSpec((ROWBLK, D), lambda i: (i, 0)),
        pl.BlockSpec((ROWBLK, NEIG), lambda i: (i, 0)),
    ],
    out_specs=[
        pl.BlockSpec((ROWBLK, D), lambda i: (i, 0)),
        pl.BlockSpec((ROWBLK, 1), lambda i: (i, 0)),
        pl.BlockSpec((ROWBLK, 1), lambda i: (i, 0)),
        pl.BlockSpec((NEIG, D), lambda i: (0, 0)),
    ],
    out_shape=[
        jax.ShapeDtypeStruct((NPAD, D), jnp.float32),
        jax.ShapeDtypeStruct((N, 1), jnp.float32),
        jax.ShapeDtypeStruct((N, 1), jnp.float32),
        jax.ShapeDtypeStruct((NEIG, D), jnp.float32),
    ],
)

_tc_mid = pl.pallas_call(
    _tc_mid_body,
    grid=(GRID,),
    in_specs=[
        pl.BlockSpec((ROWBLK, D), lambda i: (i, 0)),
        pl.BlockSpec((ROWBLK, 1), lambda i: (i, 0)),
        _const_spec((NEIG, D)),
        _const_spec((NEIG,)),
        _const_spec((1, 64)),
        _const_spec((64,)),
        _const_spec((64, 64)),
        _const_spec((64,)),
        _const_spec((64, 1)),
        _const_spec((1,)),
        _const_spec((4 * D, D)),
    ],
    out_specs=[
        pl.BlockSpec((ROWBLK, D), lambda i: (i, 0)),
        pl.BlockSpec((NEIG, D), lambda i: (0, 0)),
    ],
    out_shape=[
        jax.ShapeDtypeStruct((NPAD, D), jnp.float32),
        jax.ShapeDtypeStruct((NEIG, D), jnp.float32),
    ],
)

_tc_final = pl.pallas_call(
    _tc_final_body,
    grid=(GRID,),
    in_specs=[
        pl.BlockSpec((ROWBLK, D), lambda i: (i, 0)),
        pl.BlockSpec((ROWBLK, D), lambda i: (i, 0)),
        pl.BlockSpec((ROWBLK, D), lambda i: (i, 0)),
        pl.BlockSpec((ROWBLK, 1), lambda i: (i, 0)),
        pl.BlockSpec((ROWBLK, NEIG), lambda i: (i, 0)),
        _const_spec((NEIG, D)),
        _const_spec((4 * D, D)),
        _const_spec((D,)),
    ],
    out_specs=pl.BlockSpec((ROWBLK, D), lambda i: (i, 0)),
    out_shape=jax.ShapeDtypeStruct((N, D), jnp.float32),
)


def kernel(feature, edge_index, evecs, evals, W1, b1, W2, b2, W3, b3, Wlin,
           blin):
    src = edge_index[0]
    dst = edge_index[1]
    trash = jnp.full((E_HOP - E,), TRASH, jnp.int32)
    src_h = jnp.concatenate([src, trash]).reshape(NTILES, HCH, B)
    dst_h = jnp.concatenate([dst, trash]).reshape(NTILES, HCH, B)

    deg_tab = _sc_degree(dst_h)
    y0, s, dinv, g_raw = _tc_prep(feature, deg_tab, evecs)
    a1 = _sc_hop(y0, src_h, dst_h)
    y1, g2 = _tc_mid(a1, dinv, g_raw, evals, W1, b1, W2, b2, W3, b3, Wlin)
    a2 = _sc_hop(y1, src_h, dst_h)
    out = _tc_final(feature, a1, a2, s, evecs, g2, Wlin, blin)
    return out


# speedup vs baseline: 2.1996x; 2.1996x over previous
import functools

import jax
import jax.numpy as jnp
from jax import lax
from jax.experimental import pallas as pl
from jax.experimental.pallas import tpu as pltpu
from jax.experimental.pallas import tpu_sc as plsc

N = 10000
D = 128
NEIG = 32
NPAD = 10240
TRASH = N
NTILES = 16
NW = 32
HALF = NPAD // 2
HROWS = HALF + 128

E = 320000
B = 128
HCH = 160
HPW = HCH * B
E_HOP = NTILES * HPW
ROWBLK = 400
GRID = N // ROWBLK

_mesh = plsc.VectorSubcoreMesh(core_axis_name="c", subcore_axis_name="s")


@functools.partial(
    pl.kernel,
    mesh=_mesh,
    out_type=jax.ShapeDtypeStruct((NPAD, D), jnp.float32),
    scratch_types=[
        pltpu.VMEM((HCH, B), jnp.int32),
        pltpu.VMEM((B, D), jnp.float32),
        pltpu.VMEM_SHARED((HROWS, D), jnp.float32),
    ],
)
def _sc_degree(dst_hbm, deg_out, dst_v, const_v, acc):
    c = lax.axis_index("c")
    s = lax.axis_index("s")

    def fill(val):
        def row(i, _):
            for k in range(D // 16):
                const_v[i, pl.ds(k * 16, 16)] = jnp.full((16,), val,
                                                         jnp.float32)
            return 0
        lax.fori_loop(0, B, row, 0)

    fill(0.0)
    base = pl.multiple_of(s * (HALF // NTILES), 8)
    for k in range(HALF // NTILES // B):
        pltpu.sync_copy(const_v, acc.at[pl.ds(base + k * B, B)])
    for k in range(HALF // NTILES % B // 64):
        pltpu.sync_copy(const_v.at[pl.ds(0, 64)],
                        acc.at[pl.ds(base + (HALF // NTILES // B) * B
                                     + k * 64, 64)])


    fill(1.0)

    pltpu.sync_copy(dst_hbm.at[s], dst_v)

    lo = c * HALF
    def remap(j, _):
        r = j // (B // 16)
        k = j % (B // 16)
        v = dst_v[r, pl.ds(k * 16, 16)]
        local = v - lo
        ok = jnp.logical_and(local >= 0, local < HALF)
        spread = HALF + jnp.bitwise_and(v, 127)
        dst_v[r, pl.ds(k * 16, 16)] = jnp.where(ok, local, spread)
        return 0
    lax.fori_loop(0, HCH * (B // 16), remap, 0)
    plsc.subcore_barrier()

    def body(j, _):
        pltpu.sync_copy(const_v, acc.at[dst_v.at[j]], add=True)
        return 0
    lax.fori_loop(0, HCH, body, 0)
    plsc.subcore_barrier()

    obase = pl.multiple_of(lo + base, 8)
    pltpu.sync_copy(acc.at[pl.ds(base, HALF // NTILES)],
                    deg_out.at[pl.ds(obase, HALF // NTILES)])


@functools.partial(
    pl.kernel,
    mesh=_mesh,
    out_type=jax.ShapeDtypeStruct((NPAD, D), jnp.float32),
    scratch_types=[
        pltpu.VMEM((HCH, B), jnp.int32),
        pltpu.VMEM((HCH, B), jnp.int32),
        pltpu.VMEM((B, D), jnp.float32),
        pltpu.VMEM((B, D), jnp.float32),
        pltpu.VMEM_SHARED((HROWS, D), jnp.float32),
        pltpu.SemaphoreType.DMA,
        pltpu.SemaphoreType.DMA,
    ],
)
def _sc_hop(y_hbm, src_hbm, dst_hbm, a_out, src_v, dst_v, r0, r1,
            acc, s0, s1):
    c = lax.axis_index("c")
    s = lax.axis_index("s")
    bufs = (r0, r1)
    sems = (s0, s1)

    def zrow(i, _):
        for k in range(D // 16):
            r0[i, pl.ds(k * 16, 16)] = jnp.zeros((16,), jnp.float32)
        return 0
    lax.fori_loop(0, B, zrow, 0)
    base = pl.multiple_of(s * (HALF // NTILES), 8)
    for k in range(HALF // NTILES // B):
        pltpu.sync_copy(r0, acc.at[pl.ds(base + k * B, B)])
    for k in range(HALF // NTILES % B // 64):
        pltpu.sync_copy(r0.at[pl.ds(0, 64)],
                        acc.at[pl.ds(base + (HALF // NTILES // B) * B
                                     + k * 64, 64)])


    pltpu.sync_copy(src_hbm.at[s], src_v)
    pltpu.sync_copy(dst_hbm.at[s], dst_v)

    lo = c * HALF
    def remap(j, _):
        r = j // (B // 16)
        k = j % (B // 16)
        v = dst_v[r, pl.ds(k * 16, 16)]
        local = v - lo
        ok = jnp.logical_and(local >= 0, local < HALF)
        spread = HALF + jnp.bitwise_and(v, 127)
        dst_v[r, pl.ds(k * 16, 16)] = jnp.where(ok, local, spread)
        return 0
    lax.fori_loop(0, HCH * (B // 16), remap, 0)
    plsc.subcore_barrier()

    def body(j, _):
        pltpu.async_copy(y_hbm.at[src_v.at[j]], bufs[0], sems[0]).wait()
        pltpu.sync_copy(bufs[0], acc.at[dst_v.at[j]], add=True)
        return 0
    lax.fori_loop(0, HCH, body, 0)
    plsc.subcore_barrier()

    obase = pl.multiple_of(lo + base, 8)
    pltpu.sync_copy(acc.at[pl.ds(base, HALF // NTILES)],
                    a_out.at[pl.ds(obase, HALF // NTILES)])


def _tc_prep_body(f_ref, dp_ref, ev_ref, y0_ref, s_ref, dinv_ref, g_ref):
    i = pl.program_id(0)
    dp = dp_ref[...]
    deg = dp[:, 0]
    degc = jnp.maximum(deg, 1.0)
    s = lax.rsqrt(degc)
    s_ref[...] = s[:, None]
    dinv_ref[...] = (1.0 / degc)[:, None]
    x = f_ref[...]
    y0_ref[...] = x * s[:, None]

    @pl.when(i == 0)
    def _():
        g_ref[...] = jnp.zeros_like(g_ref)

    g_ref[...] += lax.dot_general(ev_ref[...], x, (((0,), (0,)), ((), ())),
                                  preferred_element_type=jnp.float32)


def _tc_mid_body(a_ref, dinv_ref, g_ref, evals_ref, w1_ref, b1_ref, w2_ref,
                 b2_ref, w3_ref, b3_ref, wlin_ref, y1_ref, g2_ref):
    a = a_ref[...]
    di = dinv_ref[...]
    y1_ref[...] = -(a * di)

    ev = evals_ref[...].reshape(NEIG, 1)
    t = jnp.maximum(jnp.dot(ev, w1_ref[...],
                            preferred_element_type=jnp.float32)
                    + b1_ref[...][None, :], 0.0)
    t = jnp.maximum(jnp.dot(t, w2_ref[...],
                            preferred_element_type=jnp.float32)
                    + b2_ref[...][None, :], 0.0)
    filt = jnp.dot(t, w3_ref[...],
                   preferred_element_type=jnp.float32) + b3_ref[...][None, :]
    g2_ref[...] = jnp.dot(filt * g_ref[...], wlin_ref[3 * D:, :],
                          preferred_element_type=jnp.float32)


def _tc_final_body(f_ref, a1_ref, a2_ref, s_ref, ev_ref, g2_ref, wlin_ref,
                   blin_ref, out_ref):
    x = f_ref[...]
    sc = s_ref[...]
    x1 = -sc * a1_ref[...]
    x2 = -2.0 * sc * a2_ref[...] - x
    wl = wlin_ref[...]
    acc = jnp.dot(x, wl[:D], preferred_element_type=jnp.float32)
    acc += jnp.dot(x1, wl[D:2 * D], preferred_element_type=jnp.float32)
    acc += jnp.dot(x2, wl[2 * D:3 * D], preferred_element_type=jnp.float32)
    acc += jnp.dot(ev_ref[...], g2_ref[...],
                   preferred_element_type=jnp.float32)
    out_ref[...] = jnp.maximum(acc + blin_ref[...][None, :], 0.0)


def _const_spec(shape):
    n = len(shape)
    return pl.BlockSpec(shape, lambda i: (0,) * n)


_tc_prep = pl.pallas_call(
    _tc_prep_body,
    grid=(GRID,),
    in_specs=[
        pl.BlockSpec((ROWBLK, D), lambda i: (i, 0)),
        pl.BlockSpec((ROWBLK, D), lambda i: (i, 0)),
        pl.BlockSpec((ROWBLK, NEIG), lambda i: (i, 0)),
    ],
    out_specs=[
        pl.BlockSpec((ROWBLK, D), lambda i: (i, 0)),
        pl.BlockSpec((ROWBLK, 1), lambda i: (i, 0)),
        pl.BlockSpec((ROWBLK, 1), lambda i: (i, 0)),
        pl.BlockSpec((NEIG, D), lambda i: (0, 0)),
    ],
    out_shape=[
        jax.ShapeDtypeStruct((NPAD, D), jnp.float32),
        jax.ShapeDtypeStruct((N, 1), jnp.float32),
        jax.ShapeDtypeStruct((N, 1), jnp.float32),
        jax.ShapeDtypeStruct((NEIG, D), jnp.float32),
    ],
)

_tc_mid = pl.pallas_call(
    _tc_mid_body,
    grid=(GRID,),
    in_specs=[
        pl.BlockSpec((ROWBLK, D), lambda i: (i, 0)),
        pl.BlockSpec((ROWBLK, 1), lambda i: (i, 0)),
        _const_spec((NEIG, D)),
        _const_spec((NEIG,)),
        _const_spec((1, 64)),
        _const_spec((64,)),
        _const_spec((64, 64)),
        _const_spec((64,)),
        _const_spec((64, 1)),
        _const_spec((1,)),
        _const_spec((4 * D, D)),
    ],
    out_specs=[
        pl.BlockSpec((ROWBLK, D), lambda i: (i, 0)),
        pl.BlockSpec((NEIG, D), lambda i: (0, 0)),
    ],
    out_shape=[
        jax.ShapeDtypeStruct((NPAD, D), jnp.float32),
        jax.ShapeDtypeStruct((NEIG, D), jnp.float32),
    ],
)

_tc_final = pl.pallas_call(
    _tc_final_body,
    grid=(GRID,),
    in_specs=[
        pl.BlockSpec((ROWBLK, D), lambda i: (i, 0)),
        pl.BlockSpec((ROWBLK, D), lambda i: (i, 0)),
        pl.BlockSpec((ROWBLK, D), lambda i: (i, 0)),
        pl.BlockSpec((ROWBLK, 1), lambda i: (i, 0)),
        pl.BlockSpec((ROWBLK, NEIG), lambda i: (i, 0)),
        _const_spec((NEIG, D)),
        _const_spec((4 * D, D)),
        _const_spec((D,)),
    ],
    out_specs=pl.BlockSpec((ROWBLK, D), lambda i: (i, 0)),
    out_shape=jax.ShapeDtypeStruct((N, D), jnp.float32),
)


def kernel(feature, edge_index, evecs, evals, W1, b1, W2, b2, W3, b3, Wlin,
           blin):
    src = edge_index[0]
    dst = edge_index[1]
    trash = TRASH + (jnp.arange(E_HOP - E, dtype=jnp.int32) % (NPAD - N))
    src_h = jnp.concatenate([src, trash]).reshape(NTILES, HCH, B)
    dst_h = jnp.concatenate([dst, trash]).reshape(NTILES, HCH, B)

    deg_tab = _sc_degree(dst_h)
    y0, s, dinv, g_raw = _tc_prep(feature, deg_tab, evecs)
    a1 = _sc_hop(y0, src_h, dst_h)
    y1, g2 = _tc_mid(a1, dinv, g_raw, evals, W1, b1, W2, b2, W3, b3, Wlin)
    a2 = _sc_hop(y1, src_h, dst_h)
    out = _tc_final(feature, a1, a2, s, evecs, g2, Wlin, blin)
    return out


# speedup vs baseline: 2.7812x; 1.2644x over previous
import functools

import jax
import jax.numpy as jnp
from jax import lax
from jax.experimental import pallas as pl
from jax.experimental.pallas import tpu as pltpu
from jax.experimental.pallas import tpu_sc as plsc

N = 10000
D = 128
NEIG = 32
NPAD = 10240
TRASH = N
NTILES = 16
NW = 32
HALF = NPAD // 2
HROWS = HALF + 128

E = 320000
B = 128
HCH = 160
HPW = HCH * B
E_HOP = NTILES * HPW
ROWBLK = 400
GRID = N // ROWBLK

_mesh = plsc.VectorSubcoreMesh(core_axis_name="c", subcore_axis_name="s")


@functools.partial(
    pl.kernel,
    mesh=_mesh,
    out_type=jax.ShapeDtypeStruct((NPAD, D), jnp.float32),
    scratch_types=[
        pltpu.VMEM((HCH, B), jnp.int32),
        pltpu.VMEM((B, D), jnp.float32),
        pltpu.VMEM_SHARED((HROWS, D), jnp.float32),
    ],
)
def _sc_degree(dst_hbm, deg_out, dst_v, const_v, acc):
    c = lax.axis_index("c")
    s = lax.axis_index("s")

    def fill(val):
        def row(i, _):
            for k in range(D // 16):
                const_v[i, pl.ds(k * 16, 16)] = jnp.full((16,), val,
                                                         jnp.float32)
            return 0
        lax.fori_loop(0, B, row, 0)

    fill(0.0)
    base = pl.multiple_of(s * (HALF // NTILES), 8)
    for k in range(HALF // NTILES // B):
        pltpu.sync_copy(const_v, acc.at[pl.ds(base + k * B, B)])
    for k in range(HALF // NTILES % B // 64):
        pltpu.sync_copy(const_v.at[pl.ds(0, 64)],
                        acc.at[pl.ds(base + (HALF // NTILES // B) * B
                                     + k * 64, 64)])


    fill(1.0)

    pltpu.sync_copy(dst_hbm.at[s], dst_v)

    lo = c * HALF
    def remap(j, _):
        r = j // (B // 16)
        k = j % (B // 16)
        v = dst_v[r, pl.ds(k * 16, 16)]
        local = v - lo
        ok = jnp.logical_and(local >= 0, local < HALF)
        spread = HALF + jnp.bitwise_and(v, 127)
        dst_v[r, pl.ds(k * 16, 16)] = jnp.where(ok, local, spread)
        return 0
    lax.fori_loop(0, HCH * (B // 16), remap, 0)
    plsc.subcore_barrier()

    def body(j, _):
        pltpu.sync_copy(const_v, acc.at[dst_v.at[j]], add=True)
        return 0
    lax.fori_loop(0, HCH, body, 0)
    plsc.subcore_barrier()

    obase = pl.multiple_of(lo + base, 8)
    pltpu.sync_copy(acc.at[pl.ds(base, HALF // NTILES)],
                    deg_out.at[pl.ds(obase, HALF // NTILES)])


@functools.partial(
    pl.kernel,
    mesh=_mesh,
    out_type=jax.ShapeDtypeStruct((NPAD, D), jnp.float32),
    scratch_types=[
        pltpu.VMEM((HCH, B), jnp.int32),
        pltpu.VMEM((HCH, B), jnp.int32),
        pltpu.VMEM((B, D), jnp.float32),
        pltpu.VMEM((B, D), jnp.float32),
        pltpu.VMEM_SHARED((HROWS, D), jnp.float32),
        pltpu.SemaphoreType.DMA,
        pltpu.SemaphoreType.DMA,
    ],
)
def _sc_hop(y_hbm, src_hbm, dst_hbm, a_out, src_v, dst_v, r0, r1,
            acc, s0, s1):
    c = lax.axis_index("c")
    s = lax.axis_index("s")
    bufs = (r0, r1)
    sems = (s0, s1)

    def zrow(i, _):
        for k in range(D // 16):
            r0[i, pl.ds(k * 16, 16)] = jnp.zeros((16,), jnp.float32)
        return 0
    lax.fori_loop(0, B, zrow, 0)
    base = pl.multiple_of(s * (HALF // NTILES), 8)
    for k in range(HALF // NTILES // B):
        pltpu.sync_copy(r0, acc.at[pl.ds(base + k * B, B)])
    for k in range(HALF // NTILES % B // 64):
        pltpu.sync_copy(r0.at[pl.ds(0, 64)],
                        acc.at[pl.ds(base + (HALF // NTILES // B) * B
                                     + k * 64, 64)])


    pltpu.sync_copy(src_hbm.at[s], src_v)
    pltpu.sync_copy(dst_hbm.at[s], dst_v)

    lo = c * HALF
    def remap(j, _):
        r = j // (B // 16)
        k = j % (B // 16)
        v = dst_v[r, pl.ds(k * 16, 16)]
        local = v - lo
        ok = jnp.logical_and(local >= 0, local < HALF)
        spread = HALF + jnp.bitwise_and(v, 127)
        dst_v[r, pl.ds(k * 16, 16)] = jnp.where(ok, local, spread)
        return 0
    lax.fori_loop(0, HCH * (B // 16), remap, 0)
    plsc.subcore_barrier()

    pltpu.async_copy(y_hbm.at[src_v.at[0]], bufs[0], sems[0])

    def macro(m, _):
        for p in range(2):
            j = m * 2 + p
            pltpu.make_async_copy(y_hbm.at[src_v.at[j]], bufs[p],
                                  sems[p]).wait()

            @pl.when(j + 1 < HCH)
            def _():
                pltpu.async_copy(y_hbm.at[src_v.at[j + 1]], bufs[1 - p],
                                 sems[1 - p])

            pltpu.sync_copy(bufs[p], acc.at[dst_v.at[j]], add=True)
        return 0
    lax.fori_loop(0, HCH // 2, macro, 0)
    plsc.subcore_barrier()

    obase = pl.multiple_of(lo + base, 8)
    pltpu.sync_copy(acc.at[pl.ds(base, HALF // NTILES)],
                    a_out.at[pl.ds(obase, HALF // NTILES)])


def _tc_prep_body(f_ref, dp_ref, ev_ref, y0_ref, s_ref, dinv_ref, g_ref):
    i = pl.program_id(0)
    dp = dp_ref[...]
    deg = dp[:, 0]
    degc = jnp.maximum(deg, 1.0)
    s = lax.rsqrt(degc)
    s_ref[...] = s[:, None]
    dinv_ref[...] = (1.0 / degc)[:, None]
    x = f_ref[...]
    y0_ref[...] = x * s[:, None]

    @pl.when(i == 0)
    def _():
        g_ref[...] = jnp.zeros_like(g_ref)

    g_ref[...] += lax.dot_general(ev_ref[...], x, (((0,), (0,)), ((), ())),
                                  preferred_element_type=jnp.float32)


def _tc_mid_body(a_ref, dinv_ref, g_ref, evals_ref, w1_ref, b1_ref, w2_ref,
                 b2_ref, w3_ref, b3_ref, wlin_ref, y1_ref, g2_ref):
    a = a_ref[...]
    di = dinv_ref[...]
    y1_ref[...] = -(a * di)

    ev = evals_ref[...].reshape(NEIG, 1)
    t = jnp.maximum(jnp.dot(ev, w1_ref[...],
                            preferred_element_type=jnp.float32)
                    + b1_ref[...][None, :], 0.0)
    t = jnp.maximum(jnp.dot(t, w2_ref[...],
                            preferred_element_type=jnp.float32)
                    + b2_ref[...][None, :], 0.0)
    filt = jnp.dot(t, w3_ref[...],
                   preferred_element_type=jnp.float32) + b3_ref[...][None, :]
    g2_ref[...] = jnp.dot(filt * g_ref[...], wlin_ref[3 * D:, :],
                          preferred_element_type=jnp.float32)


def _tc_final_body(f_ref, a1_ref, a2_ref, s_ref, ev_ref, g2_ref, wlin_ref,
                   blin_ref, out_ref):
    x = f_ref[...]
    sc = s_ref[...]
    x1 = -sc * a1_ref[...]
    x2 = -2.0 * sc * a2_ref[...] - x
    wl = wlin_ref[...]
    acc = jnp.dot(x, wl[:D], preferred_element_type=jnp.float32)
    acc += jnp.dot(x1, wl[D:2 * D], preferred_element_type=jnp.float32)
    acc += jnp.dot(x2, wl[2 * D:3 * D], preferred_element_type=jnp.float32)
    acc += jnp.dot(ev_ref[...], g2_ref[...],
                   preferred_element_type=jnp.float32)
    out_ref[...] = jnp.maximum(acc + blin_ref[...][None, :], 0.0)


def _const_spec(shape):
    n = len(shape)
    return pl.BlockSpec(shape, lambda i: (0,) * n)


_tc_prep = pl.pallas_call(
    _tc_prep_body,
    grid=(GRID,),
    in_specs=[
        pl.BlockSpec((ROWBLK, D), lambda i: (i, 0)),
        pl.BlockSpec((ROWBLK, D), lambda i: (i, 0)),
        pl.BlockSpec((ROWBLK, NEIG), lambda i: (i, 0)),
    ],
    out_specs=[
        pl.BlockSpec((ROWBLK, D), lambda i: (i, 0)),
        pl.BlockSpec((ROWBLK, 1), lambda i: (i, 0)),
        pl.BlockSpec((ROWBLK, 1), lambda i: (i, 0)),
        pl.BlockSpec((NEIG, D), lambda i: (0, 0)),
    ],
    out_shape=[
        jax.ShapeDtypeStruct((NPAD, D), jnp.float32),
        jax.ShapeDtypeStruct((N, 1), jnp.float32),
        jax.ShapeDtypeStruct((N, 1), jnp.float32),
        jax.ShapeDtypeStruct((NEIG, D), jnp.float32),
    ],
)

_tc_mid = pl.pallas_call(
    _tc_mid_body,
    grid=(GRID,),
    in_specs=[
        pl.BlockSpec((ROWBLK, D), lambda i: (i, 0)),
        pl.BlockSpec((ROWBLK, 1), lambda i: (i, 0)),
        _const_spec((NEIG, D)),
        _const_spec((NEIG,)),
        _const_spec((1, 64)),
        _const_spec((64,)),
        _const_spec((64, 64)),
        _const_spec((64,)),
        _const_spec((64, 1)),
        _const_spec((1,)),
        _const_spec((4 * D, D)),
    ],
    out_specs=[
        pl.BlockSpec((ROWBLK, D), lambda i: (i, 0)),
        pl.BlockSpec((NEIG, D), lambda i: (0, 0)),
    ],
    out_shape=[
        jax.ShapeDtypeStruct((NPAD, D), jnp.float32),
        jax.ShapeDtypeStruct((NEIG, D), jnp.float32),
    ],
)

_tc_final = pl.pallas_call(
    _tc_final_body,
    grid=(GRID,),
    in_specs=[
        pl.BlockSpec((ROWBLK, D), lambda i: (i, 0)),
        pl.BlockSpec((ROWBLK, D), lambda i: (i, 0)),
        pl.BlockSpec((ROWBLK, D), lambda i: (i, 0)),
        pl.BlockSpec((ROWBLK, 1), lambda i: (i, 0)),
        pl.BlockSpec((ROWBLK, NEIG), lambda i: (i, 0)),
        _const_spec((NEIG, D)),
        _const_spec((4 * D, D)),
        _const_spec((D,)),
    ],
    out_specs=pl.BlockSpec((ROWBLK, D), lambda i: (i, 0)),
    out_shape=jax.ShapeDtypeStruct((N, D), jnp.float32),
)


def kernel(feature, edge_index, evecs, evals, W1, b1, W2, b2, W3, b3, Wlin,
           blin):
    src = edge_index[0]
    dst = edge_index[1]
    trash = TRASH + (jnp.arange(E_HOP - E, dtype=jnp.int32) % (NPAD - N))
    src_h = jnp.concatenate([src, trash]).reshape(NTILES, HCH, B)
    dst_h = jnp.concatenate([dst, trash]).reshape(NTILES, HCH, B)

    deg_tab = _sc_degree(dst_h)
    y0, s, dinv, g_raw = _tc_prep(feature, deg_tab, evecs)
    a1 = _sc_hop(y0, src_h, dst_h)
    y1, g2 = _tc_mid(a1, dinv, g_raw, evals, W1, b1, W2, b2, W3, b3, Wlin)
    a2 = _sc_hop(y1, src_h, dst_h)
    out = _tc_final(feature, a1, a2, s, evecs, g2, Wlin, blin)
    return out


# speedup vs baseline: 4.6026x; 1.6549x over previous
import functools

import jax
import jax.numpy as jnp
from jax import lax
from jax.experimental import pallas as pl
from jax.experimental.pallas import tpu as pltpu
from jax.experimental.pallas import tpu_sc as plsc

N = 10000
D = 128
NEIG = 32
NPAD = 10240
TRASH = N
NTILES = 16
NW = 32
E = 320000
B = 128
WCH = 80
WIN = 40
E_HOP = NW * WCH * B
RPT = NPAD // NTILES
ROWBLK = 400
GRID = N // ROWBLK

_mesh = plsc.VectorSubcoreMesh(core_axis_name="c", subcore_axis_name="s")


@functools.partial(
    pl.kernel,
    mesh=_mesh,
    out_type=jax.ShapeDtypeStruct((2, NPAD, D), jnp.float32),
    scratch_types=[
        pltpu.VMEM((WIN, B), jnp.int32),
        pltpu.VMEM((B, D), jnp.float32),
        pltpu.VMEM_SHARED((NPAD, D), jnp.float32),
    ],
)
def _sc_degree(dst_hbm, deg_out, dst_v, const_v, acc):
    c = lax.axis_index("c")
    s = lax.axis_index("s")
    w = c * NTILES + s

    def fill(val):
        def row(i, _):
            for k in range(D // 16):
                const_v[i, pl.ds(k * 16, 16)] = jnp.full((16,), val,
                                                         jnp.float32)
            return 0
        lax.fori_loop(0, B, row, 0)

    fill(0.0)
    base = pl.multiple_of(s * RPT, 8)
    for k in range(RPT // B):
        pltpu.sync_copy(const_v, acc.at[pl.ds(base + k * B, B)])
    fill(1.0)
    plsc.subcore_barrier()

    for wi in range(WCH // WIN):
        pltpu.sync_copy(dst_hbm.at[w, pl.ds(wi * WIN, WIN)], dst_v)

        def body(j, _):
            pltpu.sync_copy(const_v, acc.at[dst_v.at[j]], add=True)
            return 0
        lax.fori_loop(0, WIN, body, 0)
    plsc.subcore_barrier()

    pltpu.sync_copy(acc.at[pl.ds(base, RPT)],
                    deg_out.at[c, pl.ds(base, RPT)])


@functools.partial(
    pl.kernel,
    mesh=_mesh,
    out_type=jax.ShapeDtypeStruct((2, NPAD, D), jnp.float32),
    scratch_types=[
        pltpu.VMEM((WIN, B), jnp.int32),
        pltpu.VMEM((WIN, B), jnp.int32),
        pltpu.VMEM((B, D), jnp.float32),
        pltpu.VMEM((B, D), jnp.float32),
        pltpu.VMEM_SHARED((NPAD, D), jnp.float32),
        pltpu.SemaphoreType.DMA,
        pltpu.SemaphoreType.DMA,
    ],
)
def _sc_hop(y_hbm, src_hbm, dst_hbm, a_out, src_v, dst_v, r0, r1,
            acc, s0, s1):
    c = lax.axis_index("c")
    s = lax.axis_index("s")
    w = c * NTILES + s
    bufs = (r0, r1)
    sems = (s0, s1)

    def zrow(i, _):
        for k in range(D // 16):
            r0[i, pl.ds(k * 16, 16)] = jnp.zeros((16,), jnp.float32)
        return 0
    lax.fori_loop(0, B, zrow, 0)
    base = pl.multiple_of(s * RPT, 8)
    for k in range(RPT // B):
        pltpu.sync_copy(r0, acc.at[pl.ds(base + k * B, B)])
    plsc.subcore_barrier()

    for wi in range(WCH // WIN):
        pltpu.sync_copy(src_hbm.at[w, pl.ds(wi * WIN, WIN)], src_v)
        pltpu.sync_copy(dst_hbm.at[w, pl.ds(wi * WIN, WIN)], dst_v)

        pltpu.async_copy(y_hbm.at[src_v.at[0]], bufs[0], sems[0])

        def macro(m, _):
            for p in range(2):
                j = m * 2 + p
                pltpu.make_async_copy(y_hbm.at[src_v.at[j]], bufs[p],
                                      sems[p]).wait()

                @pl.when(j + 1 < WIN)
                def _():
                    pltpu.async_copy(y_hbm.at[src_v.at[j + 1]],
                                     bufs[1 - p], sems[1 - p])

                pltpu.sync_copy(bufs[p], acc.at[dst_v.at[j]], add=True)
            return 0
        lax.fori_loop(0, WIN // 2, macro, 0)
    plsc.subcore_barrier()

    pltpu.sync_copy(acc.at[pl.ds(base, RPT)],
                    a_out.at[c, pl.ds(base, RPT)])


def _tc_prep_body(f_ref, dp_ref, ev_ref, y0_ref, s_ref, dinv_ref, g_ref):
    i = pl.program_id(0)
    dp = dp_ref[...]
    deg = dp[0, :, 0] + dp[1, :, 0]
    degc = jnp.maximum(deg, 1.0)
    s = lax.rsqrt(degc)
    s_ref[...] = s[:, None]
    dinv_ref[...] = (1.0 / degc)[:, None]
    x = f_ref[...]
    y0_ref[...] = x * s[:, None]

    @pl.when(i == 0)
    def _():
        g_ref[...] = jnp.zeros_like(g_ref)

    g_ref[...] += lax.dot_general(ev_ref[...], x, (((0,), (0,)), ((), ())),
                                  preferred_element_type=jnp.float32)


def _tc_mid_body(a_ref, dinv_ref, g_ref, evals_ref, w1_ref, b1_ref, w2_ref,
                 b2_ref, w3_ref, b3_ref, wlin_ref, y1_ref, g2_ref):
    a = a_ref[0] + a_ref[1]
    di = dinv_ref[...]
    y1_ref[...] = -(a * di)

    ev = evals_ref[...].reshape(NEIG, 1)
    t = jnp.maximum(jnp.dot(ev, w1_ref[...],
                            preferred_element_type=jnp.float32)
                    + b1_ref[...][None, :], 0.0)
    t = jnp.maximum(jnp.dot(t, w2_ref[...],
                            preferred_element_type=jnp.float32)
                    + b2_ref[...][None, :], 0.0)
    filt = jnp.dot(t, w3_ref[...],
                   preferred_element_type=jnp.float32) + b3_ref[...][None, :]
    g2_ref[...] = jnp.dot(filt * g_ref[...], wlin_ref[3 * D:, :],
                          preferred_element_type=jnp.float32)


def _tc_final_body(f_ref, a1_ref, a2_ref, s_ref, ev_ref, g2_ref, wlin_ref,
                   blin_ref, out_ref):
    x = f_ref[...]
    sc = s_ref[...]
    x1 = -sc * (a1_ref[0] + a1_ref[1])
    x2 = -2.0 * sc * (a2_ref[0] + a2_ref[1]) - x
    wl = wlin_ref[...]
    acc = jnp.dot(x, wl[:D], preferred_element_type=jnp.float32)
    acc += jnp.dot(x1, wl[D:2 * D], preferred_element_type=jnp.float32)
    acc += jnp.dot(x2, wl[2 * D:3 * D], preferred_element_type=jnp.float32)
    acc += jnp.dot(ev_ref[...], g2_ref[...],
                   preferred_element_type=jnp.float32)
    out_ref[...] = jnp.maximum(acc + blin_ref[...][None, :], 0.0)


def _const_spec(shape):
    n = len(shape)
    return pl.BlockSpec(shape, lambda i: (0,) * n)


_tc_prep = pl.pallas_call(
    _tc_prep_body,
    grid=(GRID,),
    in_specs=[
        pl.BlockSpec((ROWBLK, D), lambda i: (i, 0)),
        pl.BlockSpec((2, ROWBLK, D), lambda i: (0, i, 0)),
        pl.BlockSpec((ROWBLK, NEIG), lambda i: (i, 0)),
    ],
    out_specs=[
        pl.BlockSpec((ROWBLK, D), lambda i: (i, 0)),
        pl.BlockSpec((ROWBLK, 1), lambda i: (i, 0)),
        pl.BlockSpec((ROWBLK, 1), lambda i: (i, 0)),
        pl.BlockSpec((NEIG, D), lambda i: (0, 0)),
    ],
    out_shape=[
        jax.ShapeDtypeStruct((NPAD, D), jnp.float32),
        jax.ShapeDtypeStruct((N, 1), jnp.float32),
        jax.ShapeDtypeStruct((N, 1), jnp.float32),
        jax.ShapeDtypeStruct((NEIG, D), jnp.float32),
    ],
)

_tc_mid = pl.pallas_call(
    _tc_mid_body,
    grid=(GRID,),
    in_specs=[
        pl.BlockSpec((2, ROWBLK, D), lambda i: (0, i, 0)),
        pl.BlockSpec((ROWBLK, 1), lambda i: (i, 0)),
        _const_spec((NEIG, D)),
        _const_spec((NEIG,)),
        _const_spec((1, 64)),
        _const_spec((64,)),
        _const_spec((64, 64)),
        _const_spec((64,)),
        _const_spec((64, 1)),
        _const_spec((1,)),
        _const_spec((4 * D, D)),
    ],
    out_specs=[
        pl.BlockSpec((ROWBLK, D), lambda i: (i, 0)),
        pl.BlockSpec((NEIG, D), lambda i: (0, 0)),
    ],
    out_shape=[
        jax.ShapeDtypeStruct((NPAD, D), jnp.float32),
        jax.ShapeDtypeStruct((NEIG, D), jnp.float32),
    ],
)

_tc_final = pl.pallas_call(
    _tc_final_body,
    grid=(GRID,),
    in_specs=[
        pl.BlockSpec((ROWBLK, D), lambda i: (i, 0)),
        pl.BlockSpec((2, ROWBLK, D), lambda i: (0, i, 0)),
        pl.BlockSpec((2, ROWBLK, D), lambda i: (0, i, 0)),
        pl.BlockSpec((ROWBLK, 1), lambda i: (i, 0)),
        pl.BlockSpec((ROWBLK, NEIG), lambda i: (i, 0)),
        _const_spec((NEIG, D)),
        _const_spec((4 * D, D)),
        _const_spec((D,)),
    ],
    out_specs=pl.BlockSpec((ROWBLK, D), lambda i: (i, 0)),
    out_shape=jax.ShapeDtypeStruct((N, D), jnp.float32),
)


def kernel(feature, edge_index, evecs, evals, W1, b1, W2, b2, W3, b3, Wlin,
           blin):
    src = edge_index[0]
    dst = edge_index[1]
    trash = TRASH + (jnp.arange(E_HOP - E, dtype=jnp.int32) % (NPAD - N))
    src_h = jnp.concatenate([src, trash]).reshape(NW, WCH, B)
    dst_h = jnp.concatenate([dst, trash]).reshape(NW, WCH, B)

    deg_tab = _sc_degree(dst_h)
    y0, s, dinv, g_raw = _tc_prep(feature, deg_tab, evecs)
    a1 = _sc_hop(y0, src_h, dst_h)
    y1, g2 = _tc_mid(a1, dinv, g_raw, evals, W1, b1, W2, b2, W3, b3, Wlin)
    a2 = _sc_hop(y1, src_h, dst_h)
    out = _tc_final(feature, a1, a2, s, evecs, g2, Wlin, blin)
    return out
